# Initial kernel scaffold; baseline (speedup 1.0000x reference)
import jax
import jax.numpy as jnp
from jax.experimental import pallas as pl


def kernel(x, edge_index, edge_vals, poly_logits, hp_alpha):
    raise NotImplementedError("write your pallas kernel here")


# speedup vs baseline: 1.3470x; 1.3470x over previous
import functools

import jax
import jax.numpy as jnp
from jax import lax
from jax.experimental import pallas as pl
from jax.experimental.pallas import tpu as pltpu
from jax.experimental.pallas import tpu_sc as plsc

NC = 2
NS = 16
NW = NC * NS
C = 64
NB = 64

_MESH = plsc.VectorSubcoreMesh(core_axis_name="c", subcore_axis_name="s",
                               num_cores=NC, num_subcores=NS)


def _iota16():
    return lax.iota(jnp.int32, 16)


def _worker_id():
    return lax.axis_index("c") * NS + lax.axis_index("s")


def _scalar_at(vmem_ref, i):
    return vmem_ref[pl.ds(i, 16)][0]


def _lane_reduce(vec, op):
    vals = [vec[i] for i in range(16)]
    while len(vals) > 1:
        vals = [op(vals[2 * i], vals[2 * i + 1]) for i in range(len(vals) // 2)]
    return vals[0]


def _deg_kernel(rpw, n2, rows_hbm, bounds_hbm, out_hbm, deg_v, rows_v, bnd_v,
                stg_v):
    wid = _worker_id()
    pltpu.sync_copy(bounds_hbm, bnd_v)
    nz = (rpw + 15) // 16
    zeros = jnp.zeros((16,), jnp.float32)
    for i in range(nz):
        deg_v[pl.ds(i * 16, 16)] = zeros
    b0 = _scalar_at(bnd_v, wid)
    b1 = _scalar_at(bnd_v, wid + 1)
    a0 = (b0 // C) * C
    nch = (b1 - a0 + (C - 1)) // C
    base = wid * rpw
    iot = _iota16()

    def chunk(it, carry):
        e = a0 + it * C
        pltpu.sync_copy(rows_hbm.at[pl.ds(e, C)], rows_v)
        for g in range(C // 16):
            r16 = rows_v[pl.ds(g * 16, 16)]
            lr = r16 - base
            inb = (lr >= 0) & (lr < rpw)
            lrc = jnp.minimum(jnp.maximum(lr, 0), nz * 16 - 1)
            valf = jnp.where(inb, 1.0, 0.0)
            lanev = lrc & 15
            offv = lrc - lanev
            for i in range(16):
                off = offv[i]
                d = deg_v[pl.ds(off, 16)]
                deg_v[pl.ds(off, 16)] = d + jnp.where(iot == lanev[i],
                                                      valf[i], 0.0)
        return carry

    lax.fori_loop(0, nch, chunk, 0)
    mx = deg_v[pl.ds(0, 16)]
    for i in range(1, nz):
        mx = jnp.maximum(mx, deg_v[pl.ds(i * 16, 16)])
    stg_v[...] = jnp.full((16,), _lane_reduce(mx, jnp.maximum), jnp.float32)
    pltpu.sync_copy(stg_v, out_hbm.at[wid])


def _step_kernel(rpw, n2, mode, v_hbm, vp_hbm, s_hbm, rows_hbm, cols_hbm,
                 vals_hbm, bounds_hbm, dm_hbm, wlog_hbm, hp_hbm, kvec_hbm,
                 t_out, s_out, acc_v, msg_v, rows_v, cols_v, vals_v, bnd_v,
                 dm_v, wlog_v, hp_v, kv_v, sv, svp, ss, st, sem):
    wid = _worker_id()
    base = wid * rpw
    iot = _iota16()

    pltpu.sync_copy(bounds_hbm, bnd_v)
    pltpu.sync_copy(dm_hbm, dm_v)
    pltpu.sync_copy(wlog_hbm, wlog_v)
    pltpu.sync_copy(hp_hbm, hp_v)
    pltpu.sync_copy(kvec_hbm, kv_v)
    mx = dm_v[0]
    for i in range(1, NW):
        mx = jnp.maximum(mx, dm_v[i])
    dmax = _lane_reduce(mx, jnp.maximum)
    cc = (jnp.ones((16,), jnp.float32)
          / jnp.full((16,), dmax, jnp.float32))
    wv = wlog_v[...]
    we = jnp.exp(wv - _lane_reduce(wv, jnp.maximum))
    wsum = _lane_reduce(we, lambda a, b: a + b)
    winv = (jnp.ones((16,), jnp.float32)
            / jnp.full((16,), wsum, jnp.float32))
    kv = kv_v[...]
    wk1 = jnp.full((16,), _lane_reduce(
        jnp.where(iot == kv, we, jnp.zeros_like(we)),
        lambda a, b: a + b), jnp.float32) * winv
    if mode == 0:
        w0 = jnp.full((16,), we[0], jnp.float32) * winv
        hp_a = hp_v[...]
    else:
        w0 = None
        hp_a = None

    nacc = rpw * 128 // 16
    zeros = jnp.zeros((16,), jnp.float32)

    def zbody(i, carry):
        acc_v[pl.ds(i * 16, 16)] = zeros
        return carry

    lax.fori_loop(0, nacc, zbody, 0)

    b0 = _scalar_at(bnd_v, wid)
    b1 = _scalar_at(bnd_v, wid + 1)
    a0 = (b0 // C) * C
    nch = (b1 - a0 + (C - 1)) // C

    def chunk(it, carry):
        e = a0 + it * C
        pltpu.sync_copy(rows_hbm.at[pl.ds(e, C)], rows_v)
        pltpu.sync_copy(cols_hbm.at[pl.ds(e, C)], cols_v)
        pltpu.sync_copy(vals_hbm.at[pl.ds(e, C)], vals_v)
        pltpu.async_copy(v_hbm.at[cols_v], msg_v, sem).wait()
        for g in range(C // 16):
            r16 = rows_v[pl.ds(g * 16, 16)]
            va16 = vals_v[pl.ds(g * 16, 16)]
            lr = r16 - base
            inb = (lr >= 0) & (lr < rpw)
            lrc = jnp.minimum(jnp.maximum(lr, 0), rpw - 1)
            ve = jnp.where(inb, va16, jnp.zeros_like(va16))
            off16 = lrc * 128
            for i in range(16):
                gi = g * 16 + i
                rowoff = off16[i]
                vsc = ve[i]
                for j in range(8):
                    m = msg_v[gi, pl.ds(j * 16, 16)]
                    o = rowoff + (j * 16)
                    acc_v[pl.ds(o, 16)] = acc_v[pl.ds(o, 16)] + m * vsc
        return carry

    lax.fori_loop(0, nch, chunk, 0)

    rb = 0
    while rb < rpw:
        rbn = min(64, rpw - rb)
        pltpu.sync_copy(v_hbm.at[pl.ds(base + rb, rbn)], sv.at[pl.ds(0, rbn)])
        if mode != 0:
            pltpu.sync_copy(vp_hbm.at[pl.ds(base + rb, rbn)],
                            svp.at[pl.ds(0, rbn)])
            pltpu.sync_copy(s_hbm.at[pl.ds(base + rb, rbn)],
                            ss.at[pl.ds(0, rbn)])

        def pbody(r, carry):
            for j in range(8):
                lv = acc_v[pl.ds(rb * 128 + r * 128 + j * 16, 16)]
                vv = sv[r, pl.ds(j * 16, 16)]
                if mode == 0:
                    t = cc * lv - vv
                    sval = w0 * vv + wk1 * t - hp_a * (vv - (0.5 * cc) * lv)
                else:
                    vpv = svp[r, pl.ds(j * 16, 16)]
                    t = 2.0 * (cc * lv - vv) - vpv
                    sval = ss[r, pl.ds(j * 16, 16)] + wk1 * t
                    if mode == 2:
                        sval = -sval
                st[r, pl.ds(j * 16, 16)] = t
                ss[r, pl.ds(j * 16, 16)] = sval
            return carry

        lax.fori_loop(0, rbn, pbody, 0)
        if mode != 2:
            pltpu.sync_copy(st.at[pl.ds(0, rbn)],
                            t_out.at[pl.ds(base + rb, rbn)])
        pltpu.sync_copy(ss.at[pl.ds(0, rbn)], s_out.at[pl.ds(base + rb, rbn)])
        rb += rbn


def _make_step(rpw, n2, mode):
    f32, i32 = jnp.float32, jnp.int32
    outs = (jax.ShapeDtypeStruct((n2, 128), f32),
            jax.ShapeDtypeStruct((n2, 128), f32))
    scratch = [
        pltpu.VMEM((rpw * 128,), f32),
        pltpu.VMEM((C, 128), f32),
        pltpu.VMEM((C,), i32),
        pltpu.VMEM((C,), i32),
        pltpu.VMEM((C,), f32),
        pltpu.VMEM((NB,), i32),
        pltpu.VMEM((NW, 16), f32),
        pltpu.VMEM((16,), f32),
        pltpu.VMEM((16,), f32),
        pltpu.VMEM((16,), i32),
        pltpu.VMEM((64, 128), f32),
        pltpu.VMEM((64, 128), f32),
        pltpu.VMEM((64, 128), f32),
        pltpu.VMEM((64, 128), f32),
        pltpu.SemaphoreType.DMA,
    ]
    return pl.kernel(
        functools.partial(_step_kernel, rpw, n2, mode),
        out_type=outs, mesh=_MESH, scratch_types=scratch,
        name=f"cheb_step_m{mode}")


def _make_deg(rpw, n2):
    f32, i32 = jnp.float32, jnp.int32
    scratch = [
        pltpu.VMEM(((rpw + 15) // 16 * 16,), f32),
        pltpu.VMEM((C,), i32),
        pltpu.VMEM((NB,), i32),
        pltpu.VMEM((16,), f32),
    ]
    return pl.kernel(
        functools.partial(_deg_kernel, rpw, n2),
        out_type=jax.ShapeDtypeStruct((NW, 16), jnp.float32),
        mesh=_MESH, scratch_types=scratch, name="deg_max")


def kernel(x, edge_index, edge_vals, poly_logits, hp_alpha):
    n, h = x.shape
    e = edge_index.shape[1]
    assert h == 128 and poly_logits.shape[0] == 16
    kk = poly_logits.shape[0] - 1
    rpw = ((n + NW - 1) // NW + 7) // 8 * 8
    n2 = NW * rpw
    i32, f32 = jnp.int32, jnp.float32

    rows = edge_index[0]
    cols = edge_index[1]
    order = jnp.argsort(rows)
    rows_s = rows[order]
    cols_s = cols[order]
    vals_s = edge_vals[order]
    rows_p = jnp.concatenate([rows_s, jnp.full((C,), n2, i32)])
    cols_p = jnp.concatenate([cols_s, jnp.zeros((C,), i32)])
    vals_p = jnp.concatenate([vals_s, jnp.zeros((C,), f32)])
    bounds = jnp.searchsorted(rows_s,
                              (jnp.arange(NW + 1) * rpw).astype(i32)
                              ).astype(i32)
    bounds_p = jnp.concatenate([bounds, jnp.full((NB - NW - 1,), e, i32)])
    x_p = jnp.pad(x, ((0, n2 - n), (0, 0)))
    hp16 = jnp.full((16,), hp_alpha, f32)
    wlog = poly_logits.astype(f32)

    deg_fn = _make_deg(rpw, n2)
    first_fn = _make_step(rpw, n2, 0)
    mid_fn = _make_step(rpw, n2, 1)
    last_fn = _make_step(rpw, n2, 2)

    dm = deg_fn(rows_p, bounds_p)

    def kvec(k):
        return jnp.full((16,), k, i32)

    dummy = x_p
    t1, s = first_fn(x_p, dummy, dummy, rows_p, cols_p, vals_p, bounds_p, dm,
                     wlog, hp16, kvec(1))
    vp, vc = x_p, t1
    for k in range(1, kk - 1):
        vn, s = mid_fn(vc, vp, s, rows_p, cols_p, vals_p, bounds_p, dm,
                       wlog, hp16, kvec(k + 1))
        vp, vc = vc, vn
    _, out = last_fn(vc, vp, s, rows_p, cols_p, vals_p, bounds_p, dm,
                     wlog, hp16, kvec(kk))
    return out[:n]


# speedup vs baseline: 1.6279x; 1.2085x over previous
import functools

import jax
import jax.numpy as jnp
from jax import lax
from jax.experimental import pallas as pl
from jax.experimental.pallas import tpu as pltpu
from jax.experimental.pallas import tpu_sc as plsc

NC = 2
NS = 16
NW = NC * NS
C = 64
NB = 64

_MESH = plsc.VectorSubcoreMesh(core_axis_name="c", subcore_axis_name="s",
                               num_cores=NC, num_subcores=NS)


def _iota16():
    return lax.iota(jnp.int32, 16)


def _worker_id():
    return lax.axis_index("c") * NS + lax.axis_index("s")


def _scalar_at(vmem_ref, i):
    return vmem_ref[pl.ds(i, 16)][0]


def _lane_reduce(vec, op):
    vals = [vec[i] for i in range(16)]
    while len(vals) > 1:
        vals = [op(vals[2 * i], vals[2 * i + 1]) for i in range(len(vals) // 2)]
    return vals[0]


def _deg_kernel(rpw, n2, rows_hbm, bounds_hbm, out_hbm, deg_v, rows_v, bnd_v,
                stg_v):
    wid = _worker_id()
    pltpu.sync_copy(bounds_hbm, bnd_v)
    nz = (rpw + 15) // 16
    zeros = jnp.zeros((16,), jnp.float32)
    for i in range(nz):
        deg_v[pl.ds(i * 16, 16)] = zeros
    b0 = _scalar_at(bnd_v, wid)
    b1 = _scalar_at(bnd_v, wid + 1)
    a0 = (b0 // C) * C
    nch = (b1 - a0 + (C - 1)) // C
    base = wid * rpw
    iot = _iota16()

    def chunk(it, carry):
        e = a0 + it * C
        pltpu.sync_copy(rows_hbm.at[pl.ds(e, C)], rows_v)
        for g in range(C // 16):
            r16 = rows_v[pl.ds(g * 16, 16)]
            lr = r16 - base
            inb = (lr >= 0) & (lr < rpw)
            lrc = jnp.minimum(jnp.maximum(lr, 0), nz * 16 - 1)
            valf = jnp.where(inb, 1.0, 0.0)
            lanev = lrc & 15
            offv = lrc - lanev
            for i in range(16):
                off = offv[i]
                plsc.addupdate(deg_v.at[pl.ds(off, 16)],
                               jnp.where(iot == lanev[i], valf[i], 0.0))
        return carry

    lax.fori_loop(0, nch, chunk, 0)
    mx = deg_v[pl.ds(0, 16)]
    for i in range(1, nz):
        mx = jnp.maximum(mx, deg_v[pl.ds(i * 16, 16)])
    stg_v[...] = jnp.full((16,), _lane_reduce(mx, jnp.maximum), jnp.float32)
    pltpu.sync_copy(stg_v, out_hbm.at[wid])


def _step_kernel(rpw, n2, mode, v_hbm, vp_hbm, s_hbm, rows_hbm, cols_hbm,
                 vals_hbm, bounds_hbm, dm_hbm, wlog_hbm, hp_hbm, kvec_hbm,
                 t_out, s_out, acc_v, msg_v, rows_v, cols_v, vals_v, bnd_v,
                 dm_v, wlog_v, hp_v, kv_v, sv, svp, ss, st, sem):
    wid = _worker_id()
    base = wid * rpw
    iot = _iota16()

    pltpu.sync_copy(bounds_hbm, bnd_v)
    pltpu.sync_copy(dm_hbm, dm_v)
    pltpu.sync_copy(wlog_hbm, wlog_v)
    pltpu.sync_copy(hp_hbm, hp_v)
    pltpu.sync_copy(kvec_hbm, kv_v)
    mx = dm_v[0]
    for i in range(1, NW):
        mx = jnp.maximum(mx, dm_v[i])
    dmax = _lane_reduce(mx, jnp.maximum)
    cc = (jnp.ones((16,), jnp.float32)
          / jnp.full((16,), dmax, jnp.float32))
    wv = wlog_v[...]
    we = jnp.exp(wv - _lane_reduce(wv, jnp.maximum))
    wsum = _lane_reduce(we, lambda a, b: a + b)
    winv = (jnp.ones((16,), jnp.float32)
            / jnp.full((16,), wsum, jnp.float32))
    kv = kv_v[...]
    wk1 = jnp.full((16,), _lane_reduce(
        jnp.where(iot == kv, we, jnp.zeros_like(we)),
        lambda a, b: a + b), jnp.float32) * winv
    if mode == 0:
        w0 = jnp.full((16,), we[0], jnp.float32) * winv
        hp_a = hp_v[...]
    else:
        w0 = None
        hp_a = None

    nacc = rpw * 128 // 16
    zeros = jnp.zeros((16,), jnp.float32)

    def zbody(i, carry):
        acc_v[pl.ds(i * 16, 16)] = zeros
        return carry

    lax.fori_loop(0, nacc, zbody, 0)

    b0 = _scalar_at(bnd_v, wid)
    b1 = _scalar_at(bnd_v, wid + 1)
    a0 = (b0 // C) * C
    nch = (b1 - a0 + (C - 1)) // C

    def chunk(it, carry):
        e = a0 + it * C
        pltpu.sync_copy(rows_hbm.at[pl.ds(e, C)], rows_v)
        pltpu.sync_copy(cols_hbm.at[pl.ds(e, C)], cols_v)
        pltpu.sync_copy(vals_hbm.at[pl.ds(e, C)], vals_v)
        pltpu.async_copy(v_hbm.at[cols_v], msg_v, sem).wait()
        for g in range(C // 16):
            r16 = rows_v[pl.ds(g * 16, 16)]
            va16 = vals_v[pl.ds(g * 16, 16)]
            lr = r16 - base
            inb = (lr >= 0) & (lr < rpw)
            lrc = jnp.minimum(jnp.maximum(lr, 0), rpw - 1)
            ve = jnp.where(inb, va16, jnp.zeros_like(va16))
            off16 = lrc * 128
            for i in range(16):
                gi = g * 16 + i
                rowoff = off16[i]
                vsc = ve[i]
                for j in range(8):
                    m = msg_v[gi, pl.ds(j * 16, 16)]
                    o = rowoff + (j * 16)
                    plsc.addupdate(acc_v.at[pl.ds(o, 16)], m * vsc)
        return carry

    lax.fori_loop(0, nch, chunk, 0)

    rb = 0
    while rb < rpw:
        rbn = min(64, rpw - rb)
        pltpu.sync_copy(v_hbm.at[pl.ds(base + rb, rbn)], sv.at[pl.ds(0, rbn)])
        if mode != 0:
            pltpu.sync_copy(vp_hbm.at[pl.ds(base + rb, rbn)],
                            svp.at[pl.ds(0, rbn)])
            pltpu.sync_copy(s_hbm.at[pl.ds(base + rb, rbn)],
                            ss.at[pl.ds(0, rbn)])

        def pbody(r, carry):
            for j in range(8):
                lv = acc_v[pl.ds(rb * 128 + r * 128 + j * 16, 16)]
                vv = sv[r, pl.ds(j * 16, 16)]
                if mode == 0:
                    t = cc * lv - vv
                    sval = w0 * vv + wk1 * t - hp_a * (vv - (0.5 * cc) * lv)
                else:
                    vpv = svp[r, pl.ds(j * 16, 16)]
                    t = 2.0 * (cc * lv - vv) - vpv
                    sval = ss[r, pl.ds(j * 16, 16)] + wk1 * t
                    if mode == 2:
                        sval = -sval
                st[r, pl.ds(j * 16, 16)] = t
                ss[r, pl.ds(j * 16, 16)] = sval
            return carry

        lax.fori_loop(0, rbn, pbody, 0)
        if mode != 2:
            pltpu.sync_copy(st.at[pl.ds(0, rbn)],
                            t_out.at[pl.ds(base + rb, rbn)])
        pltpu.sync_copy(ss.at[pl.ds(0, rbn)], s_out.at[pl.ds(base + rb, rbn)])
        rb += rbn


def _make_step(rpw, n2, mode):
    f32, i32 = jnp.float32, jnp.int32
    outs = (jax.ShapeDtypeStruct((n2, 128), f32),
            jax.ShapeDtypeStruct((n2, 128), f32))
    scratch = [
        pltpu.VMEM((rpw * 128,), f32),
        pltpu.VMEM((C, 128), f32),
        pltpu.VMEM((C,), i32),
        pltpu.VMEM((C,), i32),
        pltpu.VMEM((C,), f32),
        pltpu.VMEM((NB,), i32),
        pltpu.VMEM((NW, 16), f32),
        pltpu.VMEM((16,), f32),
        pltpu.VMEM((16,), f32),
        pltpu.VMEM((16,), i32),
        pltpu.VMEM((64, 128), f32),
        pltpu.VMEM((64, 128), f32),
        pltpu.VMEM((64, 128), f32),
        pltpu.VMEM((64, 128), f32),
        pltpu.SemaphoreType.DMA,
    ]
    return pl.kernel(
        functools.partial(_step_kernel, rpw, n2, mode),
        out_type=outs, mesh=_MESH, scratch_types=scratch,
        name=f"cheb_step_m{mode}")


def _make_deg(rpw, n2):
    f32, i32 = jnp.float32, jnp.int32
    scratch = [
        pltpu.VMEM(((rpw + 15) // 16 * 16,), f32),
        pltpu.VMEM((C,), i32),
        pltpu.VMEM((NB,), i32),
        pltpu.VMEM((16,), f32),
    ]
    return pl.kernel(
        functools.partial(_deg_kernel, rpw, n2),
        out_type=jax.ShapeDtypeStruct((NW, 16), jnp.float32),
        mesh=_MESH, scratch_types=scratch, name="deg_max")


def kernel(x, edge_index, edge_vals, poly_logits, hp_alpha):
    n, h = x.shape
    e = edge_index.shape[1]
    assert h == 128 and poly_logits.shape[0] == 16
    kk = poly_logits.shape[0] - 1
    rpw = ((n + NW - 1) // NW + 7) // 8 * 8
    n2 = NW * rpw
    i32, f32 = jnp.int32, jnp.float32

    rows = edge_index[0]
    cols = edge_index[1]
    order = jnp.argsort(rows)
    rows_s = rows[order]
    cols_s = cols[order]
    vals_s = edge_vals[order]
    rows_p = jnp.concatenate([rows_s, jnp.full((C,), n2, i32)])
    cols_p = jnp.concatenate([cols_s, jnp.zeros((C,), i32)])
    vals_p = jnp.concatenate([vals_s, jnp.zeros((C,), f32)])
    bounds = jnp.searchsorted(rows_s,
                              (jnp.arange(NW + 1) * rpw).astype(i32)
                              ).astype(i32)
    bounds_p = jnp.concatenate([bounds, jnp.full((NB - NW - 1,), e, i32)])
    x_p = jnp.pad(x, ((0, n2 - n), (0, 0)))
    hp16 = jnp.full((16,), hp_alpha, f32)
    wlog = poly_logits.astype(f32)

    deg_fn = _make_deg(rpw, n2)
    first_fn = _make_step(rpw, n2, 0)
    mid_fn = _make_step(rpw, n2, 1)
    last_fn = _make_step(rpw, n2, 2)

    dm = deg_fn(rows_p, bounds_p)

    def kvec(k):
        return jnp.full((16,), k, i32)

    dummy = x_p
    t1, s = first_fn(x_p, dummy, dummy, rows_p, cols_p, vals_p, bounds_p, dm,
                     wlog, hp16, kvec(1))
    vp, vc = x_p, t1
    for k in range(1, kk - 1):
        vn, s = mid_fn(vc, vp, s, rows_p, cols_p, vals_p, bounds_p, dm,
                       wlog, hp16, kvec(k + 1))
        vp, vc = vc, vn
    _, out = last_fn(vc, vp, s, rows_p, cols_p, vals_p, bounds_p, dm,
                     wlog, hp16, kvec(kk))
    return out[:n]


# speedup vs baseline: 2.6701x; 1.6402x over previous
import functools

import jax
import jax.numpy as jnp
from jax import lax
from jax.experimental import pallas as pl
from jax.experimental.pallas import tpu as pltpu
from jax.experimental.pallas import tpu_sc as plsc

NC = 2
NS = 16
NW = NC * NS
C = 128


def _iota16():
    return lax.iota(jnp.int32, 16)


def _worker_id():
    return lax.axis_index("c") * NS + lax.axis_index("s")


def _scalar_at(vmem_ref, i):
    return vmem_ref[pl.ds(i, 16)][0]


def _lane_reduce(vec, op):
    vals = [vec[i] for i in range(16)]
    while len(vals) > 1:
        vals = [op(vals[2 * i], vals[2 * i + 1]) for i in range(len(vals) // 2)]
    return vals[0]


def _deg_kernel(rpw, rowptr_hbm, out_hbm, rp_v, stg_v):
    wid = _worker_id()
    base = wid * rpw
    pltpu.sync_copy(rowptr_hbm.at[pl.ds(base, rpw + 32)], rp_v)
    mx = None
    for i in range(rpw // 16):
        d = (rp_v[pl.ds(i * 16 + 1, 16)] - rp_v[pl.ds(i * 16, 16)]
             ).astype(jnp.float32)
        mx = d if mx is None else jnp.maximum(mx, d)
    stg_v[...] = jnp.full((16,), _lane_reduce(mx, jnp.maximum), jnp.float32)
    pltpu.sync_copy(stg_v, out_hbm.at[wid])


def _step_kernel(rpw, mode, v_hbm, vp_hbm, s_hbm, cols_hbm, valsw_hbm,
                 rowptr_hbm, cf_hbm, dm_hbm, wlog_hbm, hp_hbm,
                 kvec_hbm, t_out, s_out, acc_v, mb0, mb1, cb0, cb1, wb0, wb1,
                 rp_v, cf_v, dm_v, wlog_v, hp_v, kv_v, sv, svp, ss,
                 gs0, gs1, cs0, cs1, ws0, ws1):
    wid = _worker_id()
    base = wid * rpw
    iot = _iota16()

    pltpu.sync_copy(rowptr_hbm.at[pl.ds(base, rpw + 32)], rp_v)
    pltpu.sync_copy(cf_hbm, cf_v)
    pltpu.sync_copy(dm_hbm, dm_v)
    pltpu.sync_copy(wlog_hbm, wlog_v)
    pltpu.sync_copy(hp_hbm, hp_v)
    pltpu.sync_copy(kvec_hbm, kv_v)
    mx = dm_v[0]
    for i in range(1, NW):
        mx = jnp.maximum(mx, dm_v[i])
    dmax = _lane_reduce(mx, jnp.maximum)
    cc = (jnp.ones((16,), jnp.float32)
          / jnp.full((16,), dmax, jnp.float32))
    wv = wlog_v[...]
    we = jnp.exp(wv - _lane_reduce(wv, jnp.maximum))
    wsum = _lane_reduce(we, lambda a, b: a + b)
    winv = (jnp.ones((16,), jnp.float32)
            / jnp.full((16,), wsum, jnp.float32))
    kv = kv_v[...]
    wk1 = jnp.full((16,), _lane_reduce(
        jnp.where(iot == kv, we, jnp.zeros_like(we)),
        lambda a, b: a + b), jnp.float32) * winv
    if mode == 0:
        w0 = jnp.full((16,), we[0], jnp.float32) * winv
        hp_a = hp_v[...]
    else:
        w0 = None
        hp_a = None

    b0 = _scalar_at(rp_v, 0)
    b1 = _scalar_at(rp_v, rpw)

    zeros = jnp.zeros((16,), jnp.float32)

    def zbody(i, carry):
        for u in range(4):
            acc_v[pl.ds(i * 64 + u * 16, 16)] = zeros
        return carry

    lax.fori_loop(0, rpw * 128 // 64, zbody, 0)

    a0 = (b0 // C) * C
    a0c = b0 // C
    nch = jnp.maximum((b1 - a0 + (C - 1)) // C, 1)
    npairs = (nch + 1) // 2
    trips = 2 * npairs

    mbufs, cbufs, wbufs = (mb0, mb1), (cb0, cb1), (wb0, wb1)
    gsems, csems, wsems = (gs0, gs1), (cs0, cs1), (ws0, ws1)

    pltpu.sync_copy(cols_hbm.at[pl.ds(a0, C)], cb0)
    pltpu.async_copy(cols_hbm.at[pl.ds(a0 + C, C)], cb1, cs1)
    pltpu.async_copy(v_hbm.at[cb0], mb0, gs0)
    pltpu.async_copy(valsw_hbm.at[pl.ds(a0, C)], wb0, ws0)

    def pair(p, carry):
        for b in (0, 1):
            i = p * 2 + b
            e_lo = a0 + i * C
            e_hi = e_lo + C
            mb, wb = mbufs[b], wbufs[b]
            pltpu.make_async_copy(v_hbm.at[cbufs[b]], mb, gsems[b]).wait()
            pltpu.make_async_copy(valsw_hbm.at[pl.ds(e_lo, C)], wb,
                                  wsems[b]).wait()

            @pl.when(i + 2 < trips)
            def _():
                pltpu.async_copy(cols_hbm.at[pl.ds(e_lo + 2 * C, C)],
                                 cbufs[b], csems[b])

            @pl.when(i + 1 < trips)
            def _():
                pltpu.make_async_copy(
                    cols_hbm.at[pl.ds(e_lo + C, C)], cbufs[1 - b],
                    csems[1 - b]).wait()
                pltpu.async_copy(v_hbm.at[cbufs[1 - b]], mbufs[1 - b],
                                 gsems[1 - b])
                pltpu.async_copy(valsw_hbm.at[pl.ds(e_lo + C, C)],
                                 wbufs[1 - b], wsems[1 - b])

            cidx = a0c + i
            rf = jnp.minimum(jnp.maximum(_scalar_at(cf_v, cidx) - base, 0),
                             rpw)
            rl1 = jnp.minimum(_scalar_at(cf_v, cidx + 1) - base, rpw - 1) + 1
            rl1 = jnp.maximum(rl1, rf)
            p0 = _scalar_at(rp_v, rf)

            def rbody(r, pcar):
                p1 = _scalar_at(rp_v, r + 1)
                s = jnp.maximum(pcar, e_lo)
                t = jnp.maximum(jnp.minimum(p1, e_hi), s)

                def ebody(e, acc):
                    le = e - e_lo
                    va = wb[le]
                    return tuple(acc[j] + mb[le, pl.ds(j * 16, 16)] * va
                                 for j in range(8))

                acc8 = lax.fori_loop(s, t, ebody, (zeros,) * 8)
                for j in range(8):
                    plsc.addupdate(acc_v.at[pl.ds(r * 128 + j * 16, 16)],
                                   acc8[j])
                return p1

            lax.fori_loop(rf, rl1, rbody, p0)
        return carry

    lax.fori_loop(0, npairs, pair, 0)

    for blk in range(rpw // 32):
        rb = blk * 32
        pltpu.sync_copy(v_hbm.at[pl.ds(base + rb, 32)], sv)
        if mode != 0:
            pltpu.sync_copy(vp_hbm.at[pl.ds(base + rb, 32)], svp)
            pltpu.sync_copy(s_hbm.at[pl.ds(base + rb, 32)], ss)

        def pbody(r, carry):
            for j in range(8):
                lv = acc_v[pl.ds(rb * 128 + r * 128 + j * 16, 16)]
                vv = sv[r, pl.ds(j * 16, 16)]
                if mode == 0:
                    t = cc * lv - vv
                    sval = w0 * vv + wk1 * t - hp_a * (vv - (0.5 * cc) * lv)
                else:
                    vpv = svp[r, pl.ds(j * 16, 16)]
                    t = 2.0 * (cc * lv - vv) - vpv
                    sval = ss[r, pl.ds(j * 16, 16)] + wk1 * t
                    if mode == 2:
                        sval = -sval
                sv[r, pl.ds(j * 16, 16)] = t
                ss[r, pl.ds(j * 16, 16)] = sval
            return carry

        lax.fori_loop(0, 32, pbody, 0)
        if mode != 2:
            pltpu.sync_copy(sv, t_out.at[pl.ds(base + rb, 32)])
        pltpu.sync_copy(ss, s_out.at[pl.ds(base + rb, 32)])


def _make_mesh():
    return plsc.VectorSubcoreMesh(core_axis_name="c", subcore_axis_name="s",
                                  num_cores=NC, num_subcores=NS)


def _make_step(rpw, n2, nck, mode):
    f32, i32 = jnp.float32, jnp.int32
    outs = (jax.ShapeDtypeStruct((n2, 128), f32),
            jax.ShapeDtypeStruct((n2, 128), f32))
    scratch = [
        pltpu.VMEM((rpw * 128,), f32),
        pltpu.VMEM((C, 128), f32),
        pltpu.VMEM((C, 128), f32),
        pltpu.VMEM((C,), i32),
        pltpu.VMEM((C,), i32),
        pltpu.VMEM((C, 16), f32),
        pltpu.VMEM((C, 16), f32),
        pltpu.VMEM((rpw + 32,), i32),
        pltpu.VMEM((nck + 48,), i32),
        pltpu.VMEM((NW, 16), f32),
        pltpu.VMEM((16,), f32),
        pltpu.VMEM((16,), f32),
        pltpu.VMEM((16,), i32),
        pltpu.VMEM((32, 128), f32),
        pltpu.VMEM((32, 128), f32),
        pltpu.VMEM((32, 128), f32),
        pltpu.SemaphoreType.DMA,
        pltpu.SemaphoreType.DMA,
        pltpu.SemaphoreType.DMA,
        pltpu.SemaphoreType.DMA,
        pltpu.SemaphoreType.DMA,
        pltpu.SemaphoreType.DMA,
    ]
    return pl.kernel(
        functools.partial(_step_kernel, rpw, mode),
        out_type=outs, mesh=_make_mesh(), scratch_types=scratch,
        name=f"cheb_step_m{mode}")


def _make_deg(rpw):
    f32, i32 = jnp.float32, jnp.int32
    scratch = [
        pltpu.VMEM((rpw + 32,), i32),
        pltpu.VMEM((16,), f32),
    ]
    return pl.kernel(
        functools.partial(_deg_kernel, rpw),
        out_type=jax.ShapeDtypeStruct((NW, 16), f32),
        mesh=_make_mesh(), scratch_types=scratch, name="deg_max")


def kernel(x, edge_index, edge_vals, poly_logits, hp_alpha):
    n, h = x.shape
    e = edge_index.shape[1]
    assert h == 128 and poly_logits.shape[0] == 16
    kk = poly_logits.shape[0] - 1
    rpw = ((n + NW - 1) // NW + 63) // 64 * 64
    n2 = NW * rpw
    epad = 3 * C
    i32, f32 = jnp.int32, jnp.float32

    rows = edge_index[0]
    cols = edge_index[1]
    order = jnp.argsort(rows)
    rows_s = rows[order]
    cols_s = cols[order]
    vals_s = edge_vals[order]
    cols_p = jnp.concatenate([cols_s, jnp.zeros((epad,), i32)])
    valsw = jnp.broadcast_to(
        jnp.concatenate([vals_s, jnp.zeros((epad,), f32)])[:, None],
        (e + epad, 16))
    rowptr = jnp.searchsorted(rows_s, jnp.arange(n2 + 1, dtype=i32)
                              ).astype(i32)
    rowptr_p = jnp.concatenate(
        [rowptr, jnp.full((64,), jnp.int32(2 ** 30), i32)])
    rows_p2 = jnp.concatenate([rows_s, jnp.full((epad,), n2, i32)])
    nck = (e + epad) // C
    cf = jnp.concatenate([rows_p2[0::C], jnp.full((48,), n2, i32)])
    x_p = jnp.pad(x, ((0, n2 - n), (0, 0)))
    hp16 = jnp.full((16,), hp_alpha, f32)
    wlog = poly_logits.astype(f32)

    deg_fn = _make_deg(rpw)
    first_fn = _make_step(rpw, n2, nck, 0)
    mid_fn = _make_step(rpw, n2, nck, 1)
    last_fn = _make_step(rpw, n2, nck, 2)

    dm = deg_fn(rowptr_p)

    def kvec(k):
        return jnp.full((16,), k, i32)

    dummy = x_p
    t1, s = first_fn(x_p, dummy, dummy, cols_p, valsw, rowptr_p, cf, dm,
                     wlog, hp16, kvec(1))
    vp, vc = x_p, t1
    for k in range(1, kk - 1):
        vn, s = mid_fn(vc, vp, s, cols_p, valsw, rowptr_p, cf, dm,
                       wlog, hp16, kvec(k + 1))
        vp, vc = vc, vn
    _, out = last_fn(vc, vp, s, cols_p, valsw, rowptr_p, cf, dm,
                     wlog, hp16, kvec(kk))
    return out[:n]
